# Initial kernel scaffold; baseline (speedup 1.0000x reference)
import jax
import jax.numpy as jnp
from jax.experimental import pallas as pl


def kernel(edges, features, W1, b1, W2, b2):
    raise NotImplementedError("write your pallas kernel here")


# speedup vs baseline: 26.8781x; 26.8781x over previous
import functools

import jax
import jax.numpy as jnp
from jax import lax
from jax.experimental import pallas as pl
from jax.experimental.pallas import tpu as pltpu
from jax.experimental.pallas import tpu_sc as plsc

N = 10000
D = 128
E = 320000
NC = 2
NS = 16
NW = NC * NS
L = 16
EPW = E // NW
CHUNK = 80
NCHUNK = EPW // CHUNK
NPAD = 10240
STRIPE = NPAD // NS
GRP = 25
NGRP = NCHUNK // GRP

_mesh = plsc.VectorSubcoreMesh(core_axis_name="c", subcore_axis_name="s")


@functools.partial(
    pl.kernel,
    out_type=jax.ShapeDtypeStruct((NC, NPAD), jnp.float32),
    mesh=_mesh,
    scratch_types=[
        pltpu.VMEM((GRP, CHUNK), jnp.int32),
        pltpu.VMEM((CHUNK,), jnp.float32),
        pltpu.VMEM((STRIPE,), jnp.float32),
        pltpu.VMEM_SHARED((NPAD,), jnp.float32),
    ],
)
def _deg_kernel(dst_hbm, deg_hbm, dstv, onesv, zbuf, shacc):
    c = lax.axis_index("c")
    s = lax.axis_index("s")
    w = s * NC + c

    def _zero(i, _):
        zbuf[pl.ds(i * L, L)] = jnp.zeros((L,), jnp.float32)
        return 0

    lax.fori_loop(0, STRIPE // L, _zero, 0)

    def _ones(i, _):
        onesv[pl.ds(i * L, L)] = jnp.ones((L,), jnp.float32)
        return 0

    lax.fori_loop(0, CHUNK // L, _ones, 0)

    pltpu.sync_copy(zbuf, shacc.at[pl.ds(s * STRIPE, STRIPE)])
    plsc.subcore_barrier()

    def _group(g, _):
        pltpu.sync_copy(dst_hbm.at[w, g], dstv)

        def _body(j, _):
            pltpu.sync_copy(onesv, shacc.at[dstv.at[j]], add=True)
            return 0

        lax.fori_loop(0, GRP, _body, 0)
        return 0

    lax.fori_loop(0, NGRP, _group, 0)
    plsc.subcore_barrier()
    pltpu.sync_copy(shacc.at[pl.ds(s * STRIPE, STRIPE)],
                    deg_hbm.at[c, pl.ds(s * STRIPE, STRIPE)])


@functools.partial(
    pl.kernel,
    out_type=jax.ShapeDtypeStruct((NC, NPAD, D), jnp.float32),
    mesh=_mesh,
    scratch_types=[
        pltpu.VMEM((GRP, CHUNK), jnp.int32),
        pltpu.VMEM((GRP, CHUNK), jnp.int32),
        pltpu.VMEM((CHUNK, D), jnp.float32),
        pltpu.VMEM_SHARED((NPAD, D), jnp.float32),
        pltpu.SemaphoreType.DMA,
    ],
)
def _edge_kernel(y_hbm, src_hbm, dst_hbm, acc_hbm, srcv, dstv, rows,
                 shacc, sem):
    c = lax.axis_index("c")
    s = lax.axis_index("s")
    w = s * NC + c

    def _zero(i, _):
        r = i // (D // L)
        q = i % (D // L)
        rows[r, pl.ds(q * L, L)] = jnp.zeros((L,), jnp.float32)
        return 0

    lax.fori_loop(0, CHUNK * (D // L), _zero, 0)

    def _zcopy(k, _):
        pltpu.sync_copy(rows, shacc.at[pl.ds(s * STRIPE + k * CHUNK, CHUNK)])
        return 0

    lax.fori_loop(0, STRIPE // CHUNK, _zcopy, 0)
    plsc.subcore_barrier()

    def _group(g, _):
        pltpu.sync_copy(src_hbm.at[w, g], srcv)
        pltpu.sync_copy(dst_hbm.at[w, g], dstv)

        def _body(j, _):
            pltpu.async_copy(y_hbm.at[srcv.at[j]], rows, sem).wait()
            pltpu.sync_copy(rows, shacc.at[dstv.at[j]], add=True)
            return 0

        lax.fori_loop(0, GRP, _body, 0)
        return 0

    lax.fori_loop(0, NGRP, _group, 0)
    plsc.subcore_barrier()
    pltpu.sync_copy(shacc.at[pl.ds(s * STRIPE, STRIPE)],
                    acc_hbm.at[c, pl.ds(s * STRIPE, STRIPE)])


BLK = 2000


def _scale_mm_body(x_ref, w_ref, dega_ref, degb_ref, y_ref, dinv_ref):
    deg = dega_ref[...] + degb_ref[...] + 1.0
    dinv = lax.rsqrt(deg)
    dinv_ref[...] = dinv
    xw = jnp.dot(x_ref[...], w_ref[...], preferred_element_type=jnp.float32)
    y_ref[...] = xw * dinv


def _combine_body(acc_ref, y_ref, dinv_ref, b_ref, z_ref):
    t = acc_ref[0] + acc_ref[1] + y_ref[...]
    z_ref[...] = t * dinv_ref[...] + b_ref[...]


def kernel(edges, features, W1, b1, W2, b2):
    src = edges[0].reshape(NW, NGRP, GRP, CHUNK)
    dst = edges[1].reshape(NW, NGRP, GRP, CHUNK)

    deg = _deg_kernel(dst)
    dega = deg[0].reshape(NPAD, 1)
    degb = deg[1].reshape(NPAD, 1)

    grid = N // BLK
    y, dinv = pl.pallas_call(
        _scale_mm_body,
        grid=(grid,),
        in_specs=[
            pl.BlockSpec((BLK, D), lambda j: (j, 0)),
            pl.BlockSpec((D, D), lambda j: (0, 0)),
            pl.BlockSpec((BLK, 1), lambda j: (j, 0)),
            pl.BlockSpec((BLK, 1), lambda j: (j, 0)),
        ],
        out_specs=[
            pl.BlockSpec((BLK, D), lambda j: (j, 0)),
            pl.BlockSpec((BLK, 1), lambda j: (j, 0)),
        ],
        out_shape=[
            jax.ShapeDtypeStruct((N, D), jnp.float32),
            jax.ShapeDtypeStruct((N, 1), jnp.float32),
        ],
    )(features, W2, dega, degb)

    acc = _edge_kernel(y, src, dst)

    z = pl.pallas_call(
        _combine_body,
        grid=(grid,),
        in_specs=[
            pl.BlockSpec((NC, BLK, D), lambda j: (0, j, 0)),
            pl.BlockSpec((BLK, D), lambda j: (j, 0)),
            pl.BlockSpec((BLK, 1), lambda j: (j, 0)),
            pl.BlockSpec((1, D), lambda j: (0, 0)),
        ],
        out_specs=pl.BlockSpec((BLK, D), lambda j: (j, 0)),
        out_shape=jax.ShapeDtypeStruct((N, D), jnp.float32),
    )(acc, y, dinv, b2.reshape(1, D))
    return z


# speedup vs baseline: 30.6972x; 1.1421x over previous
import functools

import jax
import jax.numpy as jnp
from jax import lax
from jax.experimental import pallas as pl
from jax.experimental.pallas import tpu as pltpu
from jax.experimental.pallas import tpu_sc as plsc

N = 10000
D = 128
E = 320000
NC = 2
NS = 16
NW = NC * NS
L = 16
EPW = E // NW
CHUNK = 40
NCHUNK = EPW // CHUNK
NPAD = 10240
STRIPE = NPAD // NS
DCH = 80
DGRP = 25
DNGRP = EPW // (DCH * DGRP)

_mesh = plsc.VectorSubcoreMesh(core_axis_name="c", subcore_axis_name="s")


@functools.partial(
    pl.kernel,
    out_type=jax.ShapeDtypeStruct((NC, NPAD), jnp.float32),
    mesh=_mesh,
    scratch_types=[
        pltpu.VMEM((DGRP, DCH), jnp.int32),
        pltpu.VMEM((DCH,), jnp.float32),
        pltpu.VMEM((STRIPE,), jnp.float32),
        pltpu.VMEM_SHARED((NPAD,), jnp.float32),
    ],
)
def _deg_kernel(dst_hbm, deg_hbm, dstv, onesv, zbuf, shacc):
    c = lax.axis_index("c")
    s = lax.axis_index("s")
    w = s * NC + c

    def _zero(i, _):
        zbuf[pl.ds(i * L, L)] = jnp.zeros((L,), jnp.float32)
        return 0

    lax.fori_loop(0, STRIPE // L, _zero, 0)

    def _ones(i, _):
        onesv[pl.ds(i * L, L)] = jnp.ones((L,), jnp.float32)
        return 0

    lax.fori_loop(0, DCH // L, _ones, 0)

    pltpu.sync_copy(zbuf, shacc.at[pl.ds(s * STRIPE, STRIPE)])
    plsc.subcore_barrier()

    def _group(g, _):
        pltpu.sync_copy(dst_hbm.at[w, g], dstv)

        def _body(j, _):
            pltpu.sync_copy(onesv, shacc.at[dstv.at[j]], add=True)
            return 0

        lax.fori_loop(0, DGRP, _body, 0)
        return 0

    lax.fori_loop(0, DNGRP, _group, 0)
    plsc.subcore_barrier()
    pltpu.sync_copy(shacc.at[pl.ds(s * STRIPE, STRIPE)],
                    deg_hbm.at[c, pl.ds(s * STRIPE, STRIPE)])


@functools.partial(
    pl.kernel,
    out_type=jax.ShapeDtypeStruct((NC, NPAD, D), jnp.float32),
    mesh=_mesh,
    scratch_types=[
        pltpu.VMEM((DGRP, CHUNK), jnp.int32),
        pltpu.VMEM((DGRP, CHUNK), jnp.int32),
        pltpu.VMEM((CHUNK, D), jnp.float32),
        pltpu.VMEM((CHUNK, D), jnp.float32),
        pltpu.VMEM_SHARED((NPAD, D), jnp.float32),
        pltpu.SemaphoreType.DMA,
        pltpu.SemaphoreType.DMA,
    ],
)
def _edge_kernel(y_hbm, src_hbm, dst_hbm, acc_hbm, srcv, dstv, rowsa, rowsb,
                 shacc, sema, semb):
    c = lax.axis_index("c")
    s = lax.axis_index("s")
    w = s * NC + c

    def _zero(i, _):
        r = i // (D // L)
        q = i % (D // L)
        rowsa[r, pl.ds(q * L, L)] = jnp.zeros((L,), jnp.float32)
        return 0

    lax.fori_loop(0, CHUNK * (D // L), _zero, 0)

    def _zcopy(k, _):
        pltpu.sync_copy(rowsa, shacc.at[pl.ds(s * STRIPE + k * CHUNK, CHUNK)])
        return 0

    lax.fori_loop(0, STRIPE // CHUNK, _zcopy, 0)
    plsc.subcore_barrier()

    def _group(g, _):
        pltpu.sync_copy(src_hbm.at[w, g], srcv)
        pltpu.sync_copy(dst_hbm.at[w, g], dstv)
        pltpu.async_copy(y_hbm.at[srcv.at[0]], rowsa, sema)

        def _pair(i, _):
            j = 2 * i
            gb = pltpu.async_copy(y_hbm.at[srcv.at[j + 1]], rowsb, semb)
            pltpu.make_async_copy(y_hbm.at[srcv.at[j]], rowsa, sema).wait()
            pltpu.sync_copy(rowsa, shacc.at[dstv.at[j]], add=True)
            pltpu.async_copy(y_hbm.at[srcv.at[j + 2]], rowsa, sema)
            gb.wait()
            pltpu.sync_copy(rowsb, shacc.at[dstv.at[j + 1]], add=True)
            return 0

        lax.fori_loop(0, DGRP // 2, _pair, 0)
        pltpu.make_async_copy(y_hbm.at[srcv.at[DGRP - 1]], rowsa, sema).wait()
        pltpu.sync_copy(rowsa, shacc.at[dstv.at[DGRP - 1]], add=True)
        return 0

    lax.fori_loop(0, NCHUNK // DGRP, _group, 0)
    plsc.subcore_barrier()
    pltpu.sync_copy(shacc.at[pl.ds(s * STRIPE, STRIPE)],
                    acc_hbm.at[c, pl.ds(s * STRIPE, STRIPE)])


BLK = 2000


def _scale_mm_body(x_ref, w_ref, dega_ref, degb_ref, y_ref, dinv_ref):
    deg = dega_ref[...] + degb_ref[...] + 1.0
    dinv = lax.rsqrt(deg)
    dinv_ref[...] = dinv
    xw = jnp.dot(x_ref[...], w_ref[...], preferred_element_type=jnp.float32)
    y_ref[...] = xw * dinv


def _combine_body(acc_ref, y_ref, dinv_ref, b_ref, z_ref):
    t = acc_ref[0] + acc_ref[1] + y_ref[...]
    z_ref[...] = t * dinv_ref[...] + b_ref[...]


def kernel(edges, features, W1, b1, W2, b2):
    src = edges[0].reshape(NW, NCHUNK // DGRP, DGRP, CHUNK)
    dst = edges[1].reshape(NW, NCHUNK // DGRP, DGRP, CHUNK)
    dst4 = edges[1].reshape(NW, DNGRP, DGRP, DCH)

    deg = _deg_kernel(dst4)
    dega = deg[0].reshape(NPAD, 1)
    degb = deg[1].reshape(NPAD, 1)

    grid = N // BLK
    y, dinv = pl.pallas_call(
        _scale_mm_body,
        grid=(grid,),
        in_specs=[
            pl.BlockSpec((BLK, D), lambda j: (j, 0)),
            pl.BlockSpec((D, D), lambda j: (0, 0)),
            pl.BlockSpec((BLK, 1), lambda j: (j, 0)),
            pl.BlockSpec((BLK, 1), lambda j: (j, 0)),
        ],
        out_specs=[
            pl.BlockSpec((BLK, D), lambda j: (j, 0)),
            pl.BlockSpec((BLK, 1), lambda j: (j, 0)),
        ],
        out_shape=[
            jax.ShapeDtypeStruct((N, D), jnp.float32),
            jax.ShapeDtypeStruct((N, 1), jnp.float32),
        ],
    )(features, W2, dega, degb)

    acc = _edge_kernel(y, src, dst)

    z = pl.pallas_call(
        _combine_body,
        grid=(grid,),
        in_specs=[
            pl.BlockSpec((NC, BLK, D), lambda j: (0, j, 0)),
            pl.BlockSpec((BLK, D), lambda j: (j, 0)),
            pl.BlockSpec((BLK, 1), lambda j: (j, 0)),
            pl.BlockSpec((1, D), lambda j: (0, 0)),
        ],
        out_specs=pl.BlockSpec((BLK, D), lambda j: (j, 0)),
        out_shape=jax.ShapeDtypeStruct((N, D), jnp.float32),
    )(acc, y, dinv, b2.reshape(1, D))
    return z


# speedup vs baseline: 33.2883x; 1.0844x over previous
import functools

import jax
import jax.numpy as jnp
from jax import lax
from jax.experimental import pallas as pl
from jax.experimental.pallas import tpu as pltpu
from jax.experimental.pallas import tpu_sc as plsc

N = 10000
D = 128
E = 320000
NC = 2
NS = 16
NW = NC * NS
L = 16
NPAD = 10240
STRIPE = NPAD // NS

CHUNK = 128
EPW = 10240
EP = NW * EPW
NCHUNK = EPW // CHUNK
GRP = 5
NG = NCHUNK // GRP

_mesh = plsc.VectorSubcoreMesh(core_axis_name="c", subcore_axis_name="s")


@functools.partial(
    pl.kernel,
    out_type=jax.ShapeDtypeStruct((NC, NPAD), jnp.float32),
    mesh=_mesh,
    scratch_types=[
        pltpu.VMEM((GRP, CHUNK), jnp.int32),
        pltpu.VMEM((CHUNK,), jnp.float32),
        pltpu.VMEM((STRIPE,), jnp.float32),
        pltpu.VMEM_SHARED((NPAD,), jnp.float32),
    ],
)
def _deg_kernel(dst_hbm, deg_hbm, dstv, onesv, zbuf, shacc):
    c = lax.axis_index("c")
    s = lax.axis_index("s")
    w = s * NC + c

    def _zero(i, _):
        zbuf[pl.ds(i * L, L)] = jnp.zeros((L,), jnp.float32)
        return 0

    lax.fori_loop(0, STRIPE // L, _zero, 0)

    def _ones(i, _):
        onesv[pl.ds(i * L, L)] = jnp.ones((L,), jnp.float32)
        return 0

    lax.fori_loop(0, CHUNK // L, _ones, 0)

    pltpu.sync_copy(zbuf, shacc.at[pl.ds(s * STRIPE, STRIPE)])
    plsc.subcore_barrier()

    def _group(g, _):
        pltpu.sync_copy(dst_hbm.at[w, g], dstv)

        def _body(j, _):
            pltpu.sync_copy(onesv, shacc.at[dstv.at[j]], add=True)
            return 0

        lax.fori_loop(0, GRP, _body, 0)
        return 0

    lax.fori_loop(0, NG, _group, 0)
    plsc.subcore_barrier()
    pltpu.sync_copy(shacc.at[pl.ds(s * STRIPE, STRIPE)],
                    deg_hbm.at[c, pl.ds(s * STRIPE, STRIPE)])


@functools.partial(
    pl.kernel,
    out_type=jax.ShapeDtypeStruct((NC, NPAD, D), jnp.float32),
    mesh=_mesh,
    scratch_types=[
        pltpu.VMEM((GRP, CHUNK), jnp.int32),
        pltpu.VMEM((GRP, CHUNK), jnp.int32),
        pltpu.VMEM((CHUNK, D), jnp.float32),
        pltpu.VMEM((CHUNK, D), jnp.float32),
        pltpu.VMEM_SHARED((NPAD, D), jnp.float32),
        pltpu.SemaphoreType.DMA,
        pltpu.SemaphoreType.DMA,
    ],
)
def _edge_kernel(y_hbm, src_hbm, dst_hbm, acc_hbm, srcv, dstv, rowsa, rowsb,
                 shacc, sema, semb):
    c = lax.axis_index("c")
    s = lax.axis_index("s")
    w = s * NC + c

    def _zero(i, _):
        r = i // (D // L)
        q = i % (D // L)
        rowsa[r, pl.ds(q * L, L)] = jnp.zeros((L,), jnp.float32)
        return 0

    lax.fori_loop(0, CHUNK * (D // L), _zero, 0)

    def _zcopy(k, _):
        pltpu.sync_copy(rowsa, shacc.at[pl.ds(s * STRIPE + k * CHUNK, CHUNK)])
        return 0

    lax.fori_loop(0, STRIPE // CHUNK, _zcopy, 0)
    plsc.subcore_barrier()

    def _group(g, _):
        pltpu.sync_copy(src_hbm.at[w, g], srcv)
        pltpu.sync_copy(dst_hbm.at[w, g], dstv)
        pltpu.async_copy(y_hbm.at[srcv.at[0]], rowsa, sema)

        def _pair(i, _):
            j = 2 * i
            gb = pltpu.async_copy(y_hbm.at[srcv.at[j + 1]], rowsb, semb)
            pltpu.make_async_copy(y_hbm.at[srcv.at[j]], rowsa, sema).wait()
            pltpu.sync_copy(rowsa, shacc.at[dstv.at[j]], add=True)
            pltpu.async_copy(y_hbm.at[srcv.at[j + 2]], rowsa, sema)
            gb.wait()
            pltpu.sync_copy(rowsb, shacc.at[dstv.at[j + 1]], add=True)
            return 0

        lax.fori_loop(0, GRP // 2, _pair, 0)
        pltpu.make_async_copy(y_hbm.at[srcv.at[GRP - 1]], rowsa, sema).wait()
        pltpu.sync_copy(rowsa, shacc.at[dstv.at[GRP - 1]], add=True)
        return 0

    lax.fori_loop(0, NG, _group, 0)
    plsc.subcore_barrier()
    pltpu.sync_copy(shacc.at[pl.ds(s * STRIPE, STRIPE)],
                    acc_hbm.at[c, pl.ds(s * STRIPE, STRIPE)])


BLK = 2000


def _scale_mm_body(x_ref, w_ref, dega_ref, degb_ref, y_ref, dinv_ref):
    deg = dega_ref[...] + degb_ref[...] + 1.0
    dinv = lax.rsqrt(deg)
    dinv_ref[...] = dinv
    xw = jnp.dot(x_ref[...], w_ref[...], preferred_element_type=jnp.float32)
    y_ref[...] = xw * dinv


def _combine_body(acc_ref, y_ref, dinv_ref, b_ref, z_ref):
    t = acc_ref[0] + acc_ref[1] + y_ref[...]
    z_ref[...] = t * dinv_ref[...] + b_ref[...]


def kernel(edges, features, W1, b1, W2, b2):
    npad_e = EP - E
    padidx = N + jnp.arange(npad_e, dtype=jnp.int32) % (NPAD - N)
    src = jnp.concatenate([edges[0], padidx]).reshape(NW, NG, GRP, CHUNK)
    dst = jnp.concatenate([edges[1], padidx]).reshape(NW, NG, GRP, CHUNK)

    deg = _deg_kernel(dst)
    dega = deg[0].reshape(NPAD, 1)
    degb = deg[1].reshape(NPAD, 1)

    grid = N // BLK
    y, dinv = pl.pallas_call(
        _scale_mm_body,
        grid=(grid,),
        in_specs=[
            pl.BlockSpec((BLK, D), lambda j: (j, 0)),
            pl.BlockSpec((D, D), lambda j: (0, 0)),
            pl.BlockSpec((BLK, 1), lambda j: (j, 0)),
            pl.BlockSpec((BLK, 1), lambda j: (j, 0)),
        ],
        out_specs=[
            pl.BlockSpec((BLK, D), lambda j: (j, 0)),
            pl.BlockSpec((BLK, 1), lambda j: (j, 0)),
        ],
        out_shape=[
            jax.ShapeDtypeStruct((N, D), jnp.float32),
            jax.ShapeDtypeStruct((N, 1), jnp.float32),
        ],
    )(features, W2, dega, degb)

    y_pad = jnp.pad(y, ((0, NPAD - N), (0, 0)))
    acc = _edge_kernel(y_pad, src, dst)

    z = pl.pallas_call(
        _combine_body,
        grid=(grid,),
        in_specs=[
            pl.BlockSpec((NC, BLK, D), lambda j: (0, j, 0)),
            pl.BlockSpec((BLK, D), lambda j: (j, 0)),
            pl.BlockSpec((BLK, 1), lambda j: (j, 0)),
            pl.BlockSpec((1, D), lambda j: (0, 0)),
        ],
        out_specs=pl.BlockSpec((BLK, D), lambda j: (j, 0)),
        out_shape=jax.ShapeDtypeStruct((N, D), jnp.float32),
    )(acc, y, dinv, b2.reshape(1, D))
    return z
